# Initial kernel scaffold; baseline (speedup 1.0000x reference)
import jax
import jax.numpy as jnp
from jax.experimental import pallas as pl


def kernel(x, adj, W_mlp, b_mlp, W1, b1, W2, b2):
    raise NotImplementedError("write your pallas kernel here")


# speedup vs baseline: 47.2692x; 47.2692x over previous
import functools

import jax
import jax.numpy as jnp
from jax import lax
from jax.experimental import pallas as pl
from jax.experimental.pallas import tpu as pltpu
from jax.experimental.pallas import tpu_sc as plsc

N = 10000
NP = 10240
E = 320000
CH = 128
NSUB = 16
E_PAD = 321536
EPT = E_PAD // NSUB
NCHUNK = EPT // CH
RPT = NP // NSUB
RPT_FULL = (RPT // CH) * CH
RPT_TAIL = RPT - RPT_FULL
LANES = 16

_MESH = lambda: plsc.VectorSubcoreMesh(core_axis_name="c", subcore_axis_name="s")


def _fill_const(ref, rows, width, val):
    @pl.loop(0, rows)
    def _(r):
        @pl.loop(0, width // LANES)
        def _(j):
            ref[r, pl.ds(j * LANES, LANES)] = jnp.full((LANES,), val, jnp.float32)


def _zero_my_slice(zsrc_v, table_sh, s):
    @pl.loop(0, RPT // CH)
    def _(j):
        pltpu.sync_copy(zsrc_v, table_sh.at[pl.ds(s * RPT + j * CH, CH)])
    if RPT_TAIL:
        pltpu.sync_copy(zsrc_v.at[pl.ds(0, RPT_TAIL)],
                        table_sh.at[pl.ds(s * RPT + RPT_FULL, RPT_TAIL)])


def _agg_body(h_hbm, srcoff_hbm, dst_hbm, acc_hbm, src_v, dst_v, rows_v,
              acc_sh, sem):
    c = lax.axis_index("c")
    s = lax.axis_index("s")

    _fill_const(rows_v, CH, 128, 0.0)
    _zero_my_slice(rows_v, acc_sh, s)
    plsc.subcore_barrier()

    tile_base = s * EPT

    @pl.loop(0, NCHUNK)
    def _(i):
        base = tile_base + i * CH
        pltpu.sync_copy(srcoff_hbm.at[pl.ds(c * E_PAD + base, CH)], src_v)
        pltpu.sync_copy(dst_hbm.at[pl.ds(base, CH)], dst_v)
        pltpu.async_copy(h_hbm.at[src_v], rows_v, sem).wait()
        pltpu.sync_copy(rows_v, acc_sh.at[dst_v], add=True)

    plsc.subcore_barrier()
    pltpu.sync_copy(acc_sh.at[pl.ds(s * RPT, RPT)],
                    acc_hbm.at[pl.ds(c * NP + s * RPT, RPT)])


def _make_agg():
    scratch = [
        pltpu.VMEM((CH,), jnp.int32),
        pltpu.VMEM((CH,), jnp.int32),
        pltpu.VMEM((CH, 128), jnp.float32),
        pltpu.VMEM_SHARED((NP, 128), jnp.float32),
        pltpu.SemaphoreType.DMA,
    ]
    return pl.kernel(
        _agg_body,
        out_type=jax.ShapeDtypeStruct((2 * NP, 128), jnp.float32),
        mesh=_MESH(),
        scratch_types=scratch,
    )


def _deg_body(dst_hbm, deg_hbm, dst_v, ones_v, deg_sh):
    c = lax.axis_index("c")
    s = lax.axis_index("s")

    _fill_const(ones_v, CH, 128, 0.0)
    _zero_my_slice(ones_v, deg_sh, s)
    _fill_const(ones_v, CH, 128, 1.0)
    plsc.subcore_barrier()

    tile_base = s * EPT

    @pl.loop(0, NCHUNK)
    def _(i):
        base = tile_base + i * CH
        pltpu.sync_copy(dst_hbm.at[pl.ds(base, CH)], dst_v)
        pltpu.sync_copy(ones_v, deg_sh.at[dst_v], add=True)

    plsc.subcore_barrier()
    half = NP // 2
    rph = half // NSUB
    off = c * half + s * rph
    pltpu.sync_copy(deg_sh.at[pl.ds(off, rph)], deg_hbm.at[pl.ds(off, rph)])


def _make_deg():
    scratch = [
        pltpu.VMEM((CH,), jnp.int32),
        pltpu.VMEM((CH, 128), jnp.float32),
        pltpu.VMEM_SHARED((NP, 128), jnp.float32),
    ]
    return pl.kernel(
        _deg_body,
        out_type=jax.ShapeDtypeStruct((NP, 128), jnp.float32),
        mesh=_MESH(),
        scratch_types=scratch,
    )


def _mlp_body(x_ref, w_ref, b_ref, o_ref):
    h = jnp.dot(x_ref[0], w_ref[...], preferred_element_type=jnp.float32)
    o_ref[0] = jnp.maximum(h + b_ref[...], 0.0)


def _layer_body(relu, a_ref, d_ref, w_ref, b_ref, o_ref):
    d = jnp.maximum(d_ref[:, :1], 1.0)
    a = a_ref[0] / d
    h = jnp.dot(a, w_ref[...], preferred_element_type=jnp.float32) + b_ref[...]
    o_ref[0] = jnp.maximum(h, 0.0) if relu else h


BR = 400


def _mlp_call(x2, w, b2d):
    return pl.pallas_call(
        _mlp_body,
        grid=(2, N // BR),
        in_specs=[
            pl.BlockSpec((1, BR, x2.shape[-1]), lambda b, i: (b, i, 0)),
            pl.BlockSpec((x2.shape[-1], 128), lambda b, i: (0, 0)),
            pl.BlockSpec((1, 128), lambda b, i: (0, 0)),
        ],
        out_specs=pl.BlockSpec((1, BR, 128), lambda b, i: (b, i, 0)),
        out_shape=jax.ShapeDtypeStruct((2, N, 128), jnp.float32),
    )(x2, w, b2d)


def _layer_call(acc, degtab, w, b2d, relu):
    return pl.pallas_call(
        functools.partial(_layer_body, relu),
        grid=(2, N // BR),
        in_specs=[
            pl.BlockSpec((1, BR, 128), lambda b, i: (b, i, 0)),
            pl.BlockSpec((BR, 128), lambda b, i: (i, 0)),
            pl.BlockSpec((128, 128), lambda b, i: (0, 0)),
            pl.BlockSpec((1, 128), lambda b, i: (0, 0)),
        ],
        out_specs=pl.BlockSpec((1, BR, 128), lambda b, i: (b, i, 0)),
        out_shape=jax.ShapeDtypeStruct((2, N, 128), jnp.float32),
    )(acc, degtab, w, b2d)


def kernel(x, adj, W_mlp, b_mlp, W1, b1, W2, b2):
    b, n, l, d = x.shape
    x2 = x.reshape(b, n, l * d)
    src = adj[0].astype(jnp.int32)
    dst = adj[1].astype(jnp.int32)
    pad = E_PAD - src.shape[0]
    src_p = jnp.concatenate([src, jnp.zeros((pad,), jnp.int32)])
    dst_p = jnp.concatenate([dst, jnp.full((pad,), n, jnp.int32)])
    srcoff = jnp.concatenate([src_p, src_p + n])

    degtab = _make_deg()(dst_p)
    h = _mlp_call(x2, W_mlp, b_mlp.reshape(1, -1))
    acc1 = _make_agg()(h.reshape(b * n, 128), srcoff, dst_p)
    h1 = _layer_call(acc1.reshape(2, NP, 128), degtab, W1, b1.reshape(1, -1),
                     relu=True)
    acc2 = _make_agg()(h1.reshape(b * n, 128), srcoff, dst_p)
    out = _layer_call(acc2.reshape(2, NP, 128), degtab, W2, b2.reshape(1, -1),
                      relu=False)
    return out


# speedup vs baseline: 57.8850x; 1.2246x over previous
import functools

import jax
import jax.numpy as jnp
from jax import lax
from jax.experimental import pallas as pl
from jax.experimental.pallas import tpu as pltpu
from jax.experimental.pallas import tpu_sc as plsc

N = 10000
NP = 10240
E = 320000
CH = 128
NSUB = 16
E_PAD = 323584
EPT = E_PAD // NSUB
NCHUNK = EPT // CH
RPT = NP // NSUB
RPT_FULL = (RPT // CH) * CH
RPT_TAIL = RPT - RPT_FULL
LANES = 16

_MESH = lambda: plsc.VectorSubcoreMesh(core_axis_name="c", subcore_axis_name="s")


def _fill_const(ref, rows, width, val):
    @pl.loop(0, rows)
    def _(r):
        @pl.loop(0, width // LANES)
        def _(j):
            ref[r, pl.ds(j * LANES, LANES)] = jnp.full((LANES,), val, jnp.float32)


def _zero_my_slice(zsrc_v, table_sh, s):
    @pl.loop(0, RPT // CH)
    def _(j):
        pltpu.sync_copy(zsrc_v, table_sh.at[pl.ds(s * RPT + j * CH, CH)])
    if RPT_TAIL:
        pltpu.sync_copy(zsrc_v.at[pl.ds(0, RPT_TAIL)],
                        table_sh.at[pl.ds(s * RPT + RPT_FULL, RPT_TAIL)])


def _agg_body(h_hbm, srcoff_hbm, dst_hbm, acc_hbm,
              src0, dst0, rows0, src1, dst1, rows1,
              acc_sh, si0, si1, sg0, sg1):
    c = lax.axis_index("c")
    s = lax.axis_index("s")

    _fill_const(rows0, CH, 128, 0.0)
    _zero_my_slice(rows0, acc_sh, s)
    plsc.subcore_barrier()

    tb = s * EPT
    cb = c * E_PAD
    slots = ((src0, dst0, rows0, si0, sg0), (src1, dst1, rows1, si1, sg1))

    def idx_start(i, b):
        sv, dv, _, si, _ = slots[b]
        pltpu.async_copy(srcoff_hbm.at[pl.ds(cb + tb + i * CH, CH)], sv, si)
        pltpu.async_copy(dst_hbm.at[pl.ds(tb + i * CH, CH)], dv, si)

    def idx_wait(i, b):
        sv, dv, _, si, _ = slots[b]
        pltpu.make_async_copy(srcoff_hbm.at[pl.ds(cb + tb + i * CH, CH)], sv, si).wait()
        pltpu.make_async_copy(dst_hbm.at[pl.ds(tb + i * CH, CH)], dv, si).wait()

    def gather_start(b):
        sv, _, rv, _, sg = slots[b]
        pltpu.async_copy(h_hbm.at[sv], rv, sg)

    def gather_wait(b):
        sv, _, rv, _, sg = slots[b]
        pltpu.make_async_copy(h_hbm.at[sv], rv, sg).wait()

    def scatter(b):
        _, dv, rv, _, _ = slots[b]
        pltpu.sync_copy(rv, acc_sh.at[dv], add=True)

    idx_start(0, 0)
    idx_start(1, 1)
    idx_wait(0, 0)
    gather_start(0)

    @pl.loop(0, NCHUNK // 2 - 1)
    def _(k):
        i = 2 * k
        idx_wait(i + 1, 1)
        gather_start(1)
        gather_wait(0)
        scatter(0)
        idx_start(i + 2, 0)
        idx_wait(i + 2, 0)
        gather_start(0)
        gather_wait(1)
        scatter(1)
        idx_start(i + 3, 1)

    idx_wait(NCHUNK - 1, 1)
    gather_start(1)
    gather_wait(0)
    scatter(0)
    gather_wait(1)
    scatter(1)

    plsc.subcore_barrier()
    pltpu.sync_copy(acc_sh.at[pl.ds(s * RPT, RPT)],
                    acc_hbm.at[pl.ds(c * NP + s * RPT, RPT)])


def _make_agg():
    scratch = [
        pltpu.VMEM((CH,), jnp.int32),
        pltpu.VMEM((CH,), jnp.int32),
        pltpu.VMEM((CH, 128), jnp.float32),
        pltpu.VMEM((CH,), jnp.int32),
        pltpu.VMEM((CH,), jnp.int32),
        pltpu.VMEM((CH, 128), jnp.float32),
        pltpu.VMEM_SHARED((NP, 128), jnp.float32),
        pltpu.SemaphoreType.DMA,
        pltpu.SemaphoreType.DMA,
        pltpu.SemaphoreType.DMA,
        pltpu.SemaphoreType.DMA,
    ]
    return pl.kernel(
        _agg_body,
        out_type=jax.ShapeDtypeStruct((2 * NP, 128), jnp.float32),
        mesh=_MESH(),
        scratch_types=scratch,
    )


def _deg_body(dst_hbm, deg_hbm, dst_v, ones_v, deg_sh):
    c = lax.axis_index("c")
    s = lax.axis_index("s")

    _fill_const(ones_v, CH, 128, 0.0)
    _zero_my_slice(ones_v, deg_sh, s)
    _fill_const(ones_v, CH, 128, 1.0)
    plsc.subcore_barrier()

    tile_base = s * EPT

    @pl.loop(0, NCHUNK)
    def _(i):
        base = tile_base + i * CH
        pltpu.sync_copy(dst_hbm.at[pl.ds(base, CH)], dst_v)
        pltpu.sync_copy(ones_v, deg_sh.at[dst_v], add=True)

    plsc.subcore_barrier()
    half = NP // 2
    rph = half // NSUB
    off = c * half + s * rph
    pltpu.sync_copy(deg_sh.at[pl.ds(off, rph)], deg_hbm.at[pl.ds(off, rph)])


def _make_deg():
    scratch = [
        pltpu.VMEM((CH,), jnp.int32),
        pltpu.VMEM((CH, 128), jnp.float32),
        pltpu.VMEM_SHARED((NP, 128), jnp.float32),
    ]
    return pl.kernel(
        _deg_body,
        out_type=jax.ShapeDtypeStruct((NP, 128), jnp.float32),
        mesh=_MESH(),
        scratch_types=scratch,
    )


def _mlp_body(x_ref, w_ref, b_ref, o_ref):
    h = jnp.dot(x_ref[0], w_ref[...], preferred_element_type=jnp.float32)
    o_ref[0] = jnp.maximum(h + b_ref[...], 0.0)


def _layer_body(relu, a_ref, d_ref, w_ref, b_ref, o_ref):
    d = jnp.maximum(d_ref[:, :1], 1.0)
    a = a_ref[0] / d
    h = jnp.dot(a, w_ref[...], preferred_element_type=jnp.float32) + b_ref[...]
    o_ref[0] = jnp.maximum(h, 0.0) if relu else h


BR = 400


def _mlp_call(x2, w, b2d):
    return pl.pallas_call(
        _mlp_body,
        grid=(2, N // BR),
        in_specs=[
            pl.BlockSpec((1, BR, x2.shape[-1]), lambda b, i: (b, i, 0)),
            pl.BlockSpec((x2.shape[-1], 128), lambda b, i: (0, 0)),
            pl.BlockSpec((1, 128), lambda b, i: (0, 0)),
        ],
        out_specs=pl.BlockSpec((1, BR, 128), lambda b, i: (b, i, 0)),
        out_shape=jax.ShapeDtypeStruct((2, N, 128), jnp.float32),
    )(x2, w, b2d)


def _layer_call(acc, degtab, w, b2d, relu):
    return pl.pallas_call(
        functools.partial(_layer_body, relu),
        grid=(2, N // BR),
        in_specs=[
            pl.BlockSpec((1, BR, 128), lambda b, i: (b, i, 0)),
            pl.BlockSpec((BR, 128), lambda b, i: (i, 0)),
            pl.BlockSpec((128, 128), lambda b, i: (0, 0)),
            pl.BlockSpec((1, 128), lambda b, i: (0, 0)),
        ],
        out_specs=pl.BlockSpec((1, BR, 128), lambda b, i: (b, i, 0)),
        out_shape=jax.ShapeDtypeStruct((2, N, 128), jnp.float32),
    )(acc, degtab, w, b2d)


def kernel(x, adj, W_mlp, b_mlp, W1, b1, W2, b2):
    b, n, l, d = x.shape
    x2 = x.reshape(b, n, l * d)
    src = adj[0].astype(jnp.int32)
    dst = adj[1].astype(jnp.int32)
    pad = E_PAD - src.shape[0]
    src_p = jnp.concatenate([src, jnp.zeros((pad,), jnp.int32)])
    dst_p = jnp.concatenate([dst, jnp.full((pad,), n, jnp.int32)])
    srcoff = jnp.concatenate([src_p, src_p + n])

    degtab = _make_deg()(dst_p)
    h = _mlp_call(x2, W_mlp, b_mlp.reshape(1, -1))
    acc1 = _make_agg()(h.reshape(b * n, 128), srcoff, dst_p)
    h1 = _layer_call(acc1.reshape(2, NP, 128), degtab, W1, b1.reshape(1, -1),
                     relu=True)
    acc2 = _make_agg()(h1.reshape(b * n, 128), srcoff, dst_p)
    out = _layer_call(acc2.reshape(2, NP, 128), degtab, W2, b2.reshape(1, -1),
                      relu=False)
    return out


# speedup vs baseline: 60.1491x; 1.0391x over previous
import functools

import jax
import jax.numpy as jnp
from jax import lax
from jax.experimental import pallas as pl
from jax.experimental.pallas import tpu as pltpu
from jax.experimental.pallas import tpu_sc as plsc

N = 10000
NP = 10240
E = 320000
CH = 128
NSUB = 16
E_PAD = 323584
EPT = E_PAD // NSUB
NCHUNK = EPT // CH
RPT = NP // NSUB
RPT_FULL = (RPT // CH) * CH
RPT_TAIL = RPT - RPT_FULL
LANES = 16

_MESH = lambda: plsc.VectorSubcoreMesh(core_axis_name="c", subcore_axis_name="s")


def _fill_const(ref, rows, width, val):
    @pl.loop(0, rows)
    def _(r):
        @pl.loop(0, width // LANES)
        def _(j):
            ref[r, pl.ds(j * LANES, LANES)] = jnp.full((LANES,), val, jnp.float32)


def _zero_my_slice(zsrc_v, table_sh, s):
    @pl.loop(0, RPT // CH)
    def _(j):
        pltpu.sync_copy(zsrc_v, table_sh.at[pl.ds(s * RPT + j * CH, CH)])
    if RPT_TAIL:
        pltpu.sync_copy(zsrc_v.at[pl.ds(0, RPT_TAIL)],
                        table_sh.at[pl.ds(s * RPT + RPT_FULL, RPT_TAIL)])


def _agg_body(h_hbm, srcoff_hbm, dst_hbm, acc_hbm,
              src0, dst0, rows0, src1, dst1, rows1,
              acc_sh, si0, si1, sg0, sg1):
    c = lax.axis_index("c")
    s = lax.axis_index("s")

    _fill_const(rows0, CH, 128, 0.0)
    _zero_my_slice(rows0, acc_sh, s)
    plsc.subcore_barrier()

    tb = s * EPT
    cb = c * E_PAD
    slots = ((src0, dst0, rows0, si0, sg0), (src1, dst1, rows1, si1, sg1))

    def idx_start(i, b):
        sv, dv, _, si, _ = slots[b]
        pltpu.async_copy(srcoff_hbm.at[pl.ds(cb + tb + i * CH, CH)], sv, si)
        pltpu.async_copy(dst_hbm.at[pl.ds(tb + i * CH, CH)], dv, si)

    def idx_wait(i, b):
        sv, dv, _, si, _ = slots[b]
        pltpu.make_async_copy(srcoff_hbm.at[pl.ds(cb + tb + i * CH, CH)], sv, si).wait()
        pltpu.make_async_copy(dst_hbm.at[pl.ds(tb + i * CH, CH)], dv, si).wait()

    def gather_start(b):
        sv, _, rv, _, sg = slots[b]
        pltpu.async_copy(h_hbm.at[sv], rv, sg)

    def gather_wait(b):
        sv, _, rv, _, sg = slots[b]
        pltpu.make_async_copy(h_hbm.at[sv], rv, sg).wait()

    def scatter(b):
        _, dv, rv, _, _ = slots[b]
        pltpu.sync_copy(rv, acc_sh.at[dv], add=True)

    idx_start(0, 0)
    idx_start(1, 1)
    idx_wait(0, 0)
    gather_start(0)

    @pl.loop(0, NCHUNK // 2 - 1)
    def _(k):
        i = 2 * k
        idx_wait(i + 1, 1)
        gather_start(1)
        gather_wait(0)
        scatter(0)
        idx_start(i + 2, 0)
        idx_wait(i + 2, 0)
        gather_start(0)
        gather_wait(1)
        scatter(1)
        idx_start(i + 3, 1)

    idx_wait(NCHUNK - 1, 1)
    gather_start(1)
    gather_wait(0)
    scatter(0)
    gather_wait(1)
    scatter(1)

    plsc.subcore_barrier()
    pltpu.sync_copy(acc_sh.at[pl.ds(s * RPT, RPT)],
                    acc_hbm.at[pl.ds(c * NP + s * RPT, RPT)])


def _make_agg():
    scratch = [
        pltpu.VMEM((CH,), jnp.int32),
        pltpu.VMEM((CH,), jnp.int32),
        pltpu.VMEM((CH, 128), jnp.float32),
        pltpu.VMEM((CH,), jnp.int32),
        pltpu.VMEM((CH,), jnp.int32),
        pltpu.VMEM((CH, 128), jnp.float32),
        pltpu.VMEM_SHARED((NP, 128), jnp.float32),
        pltpu.SemaphoreType.DMA,
        pltpu.SemaphoreType.DMA,
        pltpu.SemaphoreType.DMA,
        pltpu.SemaphoreType.DMA,
    ]
    return pl.kernel(
        _agg_body,
        out_type=jax.ShapeDtypeStruct((2 * NP, 128), jnp.float32),
        mesh=_MESH(),
        scratch_types=scratch,
    )


def _deg_body(dst_hbm, deg_hbm, dst0, dst1, ones_v, deg_sh, si0, si1, ss0, ss1):
    c = lax.axis_index("c")
    s = lax.axis_index("s")

    _fill_const(ones_v, CH, 128, 0.0)
    _zero_my_slice(ones_v, deg_sh, s)
    _fill_const(ones_v, CH, 128, 1.0)
    plsc.subcore_barrier()

    tb = s * EPT
    slots = ((dst0, si0, ss0), (dst1, si1, ss1))

    def idx_start(i, b):
        dv, si, _ = slots[b]
        pltpu.async_copy(dst_hbm.at[pl.ds(tb + i * CH, CH)], dv, si)

    def idx_wait(i, b):
        dv, si, _ = slots[b]
        pltpu.make_async_copy(dst_hbm.at[pl.ds(tb + i * CH, CH)], dv, si).wait()

    def scatter_start(b):
        dv, _, ss = slots[b]
        pltpu.async_copy(ones_v, deg_sh.at[dv], ss, add=True)

    def scatter_wait(b):
        dv, _, ss = slots[b]
        pltpu.make_async_copy(ones_v, deg_sh.at[dv], ss).wait()

    idx_start(0, 0)
    idx_start(1, 1)

    @pl.loop(0, NCHUNK // 2 - 1)
    def _(k):
        i = 2 * k
        idx_wait(i, 0)
        scatter_start(0)
        idx_wait(i + 1, 1)
        scatter_start(1)
        scatter_wait(0)
        idx_start(i + 2, 0)
        scatter_wait(1)
        idx_start(i + 3, 1)

    idx_wait(NCHUNK - 2, 0)
    scatter_start(0)
    idx_wait(NCHUNK - 1, 1)
    scatter_start(1)
    scatter_wait(0)
    scatter_wait(1)

    plsc.subcore_barrier()
    half = NP // 2
    rph = half // NSUB
    off = c * half + s * rph
    pltpu.sync_copy(deg_sh.at[pl.ds(off, rph)], deg_hbm.at[pl.ds(off, rph)])


def _make_deg():
    scratch = [
        pltpu.VMEM((CH,), jnp.int32),
        pltpu.VMEM((CH,), jnp.int32),
        pltpu.VMEM((CH, 128), jnp.float32),
        pltpu.VMEM_SHARED((NP, 128), jnp.float32),
        pltpu.SemaphoreType.DMA,
        pltpu.SemaphoreType.DMA,
        pltpu.SemaphoreType.DMA,
        pltpu.SemaphoreType.DMA,
    ]
    return pl.kernel(
        _deg_body,
        out_type=jax.ShapeDtypeStruct((NP, 128), jnp.float32),
        mesh=_MESH(),
        scratch_types=scratch,
    )


def _mlp_body(x_ref, w_ref, b_ref, o_ref):
    h = jnp.dot(x_ref[0], w_ref[...], preferred_element_type=jnp.float32)
    o_ref[0] = jnp.maximum(h + b_ref[...], 0.0)


def _layer_body(relu, a_ref, d_ref, w_ref, b_ref, o_ref):
    d = jnp.maximum(d_ref[:, :1], 1.0)
    a = a_ref[0] / d
    h = jnp.dot(a, w_ref[...], preferred_element_type=jnp.float32) + b_ref[...]
    o_ref[0] = jnp.maximum(h, 0.0) if relu else h


BR = 400


def _mlp_call(x2, w, b2d):
    return pl.pallas_call(
        _mlp_body,
        grid=(2, N // BR),
        in_specs=[
            pl.BlockSpec((1, BR, x2.shape[-1]), lambda b, i: (b, i, 0)),
            pl.BlockSpec((x2.shape[-1], 128), lambda b, i: (0, 0)),
            pl.BlockSpec((1, 128), lambda b, i: (0, 0)),
        ],
        out_specs=pl.BlockSpec((1, BR, 128), lambda b, i: (b, i, 0)),
        out_shape=jax.ShapeDtypeStruct((2, N, 128), jnp.float32),
    )(x2, w, b2d)


def _layer_call(acc, degtab, w, b2d, relu):
    return pl.pallas_call(
        functools.partial(_layer_body, relu),
        grid=(2, N // BR),
        in_specs=[
            pl.BlockSpec((1, BR, 128), lambda b, i: (b, i, 0)),
            pl.BlockSpec((BR, 128), lambda b, i: (i, 0)),
            pl.BlockSpec((128, 128), lambda b, i: (0, 0)),
            pl.BlockSpec((1, 128), lambda b, i: (0, 0)),
        ],
        out_specs=pl.BlockSpec((1, BR, 128), lambda b, i: (b, i, 0)),
        out_shape=jax.ShapeDtypeStruct((2, N, 128), jnp.float32),
    )(acc, degtab, w, b2d)


def kernel(x, adj, W_mlp, b_mlp, W1, b1, W2, b2):
    b, n, l, d = x.shape
    x2 = x.reshape(b, n, l * d)
    src = adj[0].astype(jnp.int32)
    dst = adj[1].astype(jnp.int32)
    pad = E_PAD - src.shape[0]
    src_p = jnp.concatenate([src, jnp.zeros((pad,), jnp.int32)])
    dst_p = jnp.concatenate([dst, jnp.full((pad,), n, jnp.int32)])
    srcoff = jnp.concatenate([src_p, src_p + n])

    degtab = _make_deg()(dst_p)
    h = _mlp_call(x2, W_mlp, b_mlp.reshape(1, -1))
    acc1 = _make_agg()(h.reshape(b * n, 128), srcoff, dst_p)
    h1 = _layer_call(acc1.reshape(2, NP, 128), degtab, W1, b1.reshape(1, -1),
                     relu=True)
    acc2 = _make_agg()(h1.reshape(b * n, 128), srcoff, dst_p)
    out = _layer_call(acc2.reshape(2, NP, 128), degtab, W2, b2.reshape(1, -1),
                      relu=False)
    return out
